# baseline (speedup 1.0000x reference)
import jax
import jax.numpy as jnp
from jax.experimental import pallas as pl

_TEST = 0.0
_N = 131072
_D = 64
_NF = _N * _D // 128
_BRF = 1024


def _body(c_ref, d_ref, out_ref):
    cv = c_ref[...]
    dv = d_ref[...]

    lo = cv - dv
    hi = cv + dv
    left_mask = lo < _TEST
    right_mask = hi >= _TEST

    lc = (lo + jnp.minimum(hi, _TEST)) * 0.5
    ld = (jnp.minimum(lc + dv, _TEST) - (lc - dv)) * 0.5
    rc = (jnp.maximum(lo, _TEST) + hi) * 0.5
    rd = (rc + dv - jnp.maximum(rc - dv, _TEST)) * 0.5

    both = left_mask & right_mask
    j_lo = jnp.minimum(lc - ld, rc - rd)
    j_hi = jnp.maximum(lc + ld, rc + rd)
    jc = (j_lo + j_hi) * 0.5
    jd = (j_hi - j_lo) * 0.5

    ntc = jnp.where(both, jc, jnp.where(left_mask, lc, jnp.where(right_mask, rc, cv)))
    ntd = jnp.where(both, jd, jnp.where(left_mask, ld, jnp.where(right_mask, rd, dv)))

    lane = jax.lax.broadcasted_iota(jnp.int32, (1, 128), 1)
    tgt = (lane == 0) | (lane == _D)
    out_ref[0] = jnp.where(tgt, ntc, cv)
    out_ref[1] = jnp.where(tgt, ntd, dv)


def kernel(c, delta):
    cf = c.reshape(_NF, 128)
    df = delta.reshape(_NF, 128)
    out = pl.pallas_call(
        _body,
        grid=(_NF // _BRF,),
        in_specs=[
            pl.BlockSpec((_BRF, 128), lambda i: (i, 0)),
            pl.BlockSpec((_BRF, 128), lambda i: (i, 0)),
        ],
        out_specs=pl.BlockSpec((2, _BRF, 128), lambda i: (0, i, 0)),
        out_shape=jax.ShapeDtypeStruct((2, _NF, 128), jnp.float32),
    )(cf, df)
    return out.reshape(2, _N, _D)


# speedup vs baseline: 1.4593x; 1.4593x over previous
import jax
import jax.numpy as jnp
from jax.experimental import pallas as pl
from jax.experimental.pallas import tpu as pltpu

_N = 131072
_D = 64
_BR = 8192


def _body(c_ref, d_ref, out_ref):
    cv = c_ref[...]
    dv = d_ref[...]
    straddle = ((cv - dv) < 0.0) & ((cv + dv) >= 0.0)
    col0 = jax.lax.broadcasted_iota(jnp.int32, (1, _D), 1) == 0
    sel = straddle & col0
    out_ref[0] = jnp.where(sel, 0.75 * cv, cv)
    out_ref[1] = jnp.where(sel, 1.25 * dv, dv)


def kernel(c, delta):
    return pl.pallas_call(
        _body,
        grid=(_N // _BR,),
        in_specs=[
            pl.BlockSpec((_BR, _D), lambda i: (i, 0)),
            pl.BlockSpec((_BR, _D), lambda i: (i, 0)),
        ],
        out_specs=pl.BlockSpec((2, _BR, _D), lambda i: (0, i, 0)),
        out_shape=jax.ShapeDtypeStruct((2, _N, _D), jnp.float32),
        compiler_params=pltpu.CompilerParams(
            dimension_semantics=("arbitrary",),
        ),
    )(c, delta)


# speedup vs baseline: 9.0418x; 6.1959x over previous
import jax
import jax.numpy as jnp
from jax.experimental import pallas as pl
from jax.experimental.pallas import tpu as pltpu

_N = 131072
_D = 64
_BC = 8192


def _body(c_ref, d_ref, out_ref):
    cv = c_ref[...]
    dv = d_ref[...]
    straddle = ((cv - dv) < 0.0) & ((cv + dv) >= 0.0)
    row0 = jax.lax.broadcasted_iota(jnp.int32, (_D, 1), 0) == 0
    sel = straddle & row0
    out_ref[0] = jnp.where(sel, 0.75 * cv, cv)
    out_ref[1] = jnp.where(sel, 1.25 * dv, dv)


def kernel(c, delta):
    ct = c.T
    dt = delta.T
    out_t = pl.pallas_call(
        _body,
        grid=(_N // _BC,),
        in_specs=[
            pl.BlockSpec((_D, _BC), lambda i: (0, i)),
            pl.BlockSpec((_D, _BC), lambda i: (0, i)),
        ],
        out_specs=pl.BlockSpec((2, _D, _BC), lambda i: (0, 0, i)),
        out_shape=jax.ShapeDtypeStruct((2, _D, _N), jnp.float32),
        compiler_params=pltpu.CompilerParams(
            dimension_semantics=("arbitrary",),
        ),
    )(ct, dt)
    return out_t.transpose(0, 2, 1)


# speedup vs baseline: 9.3767x; 1.0370x over previous
import jax
import jax.numpy as jnp
from jax.experimental import pallas as pl
from jax.experimental.pallas import tpu as pltpu

_N = 131072
_D = 64
_BC = 16384


def _body(c_ref, d_ref, out_ref):
    cv = c_ref[...]
    dv = d_ref[...]
    straddle = ((cv - dv) < 0.0) & ((cv + dv) >= 0.0)
    row0 = jax.lax.broadcasted_iota(jnp.int32, (_D, 1), 0) == 0
    sel = straddle & row0
    out_ref[0] = jnp.where(sel, 0.75 * cv, cv)
    out_ref[1] = jnp.where(sel, 1.25 * dv, dv)


def kernel(c, delta):
    ct = c.T
    dt = delta.T
    out_t = pl.pallas_call(
        _body,
        grid=(_N // _BC,),
        in_specs=[
            pl.BlockSpec((_D, _BC), lambda i: (0, i)),
            pl.BlockSpec((_D, _BC), lambda i: (0, i)),
        ],
        out_specs=pl.BlockSpec((2, _D, _BC), lambda i: (0, 0, i)),
        out_shape=jax.ShapeDtypeStruct((2, _D, _N), jnp.float32),
        compiler_params=pltpu.CompilerParams(
            dimension_semantics=("arbitrary",),
        ),
    )(ct, dt)
    return out_t.transpose(0, 2, 1)
